# baseline (speedup 1.0000x reference)
import functools

import jax
import jax.numpy as jnp
from jax.experimental import pallas as pl

_LANES = 128


def _expand_body(idx_ref, rep_ref, src_ref, w_ref, b_ref, out_ref, pooled_ref,
                 *, seq: int, dim: int):
    flat = seq * dim
    flatp = rep_ref.shape[1]
    bt = idx_ref.shape[0]
    idx_bf = idx_ref[...].astype(jnp.bfloat16)
    rep = jnp.dot(idx_bf, rep_ref[...], preferred_element_type=jnp.float32)
    repi = rep.astype(jnp.int32)
    src = jnp.broadcast_to(src_ref[...], (bt, _LANES))
    ft = None
    for j0 in range(0, flatp, _LANES):
        kcol = jax.lax.broadcasted_iota(jnp.int32, (bt, _LANES), 1) + j0
        kcol = kcol - dim * (kcol // dim)
        lookup = repi[:, j0:j0 + _LANES] * dim + kcol
        vals = jnp.take_along_axis(src, lookup, axis=1)
        w = min(_LANES, flat - j0)
        out_ref[:, j0:j0 + w] = vals[:, :w]
        if j0 == 0:
            ft = vals[:, :dim]
    pooled = jnp.dot(ft, w_ref[...], preferred_element_type=jnp.float32)
    pooled_ref[...] = jnp.tanh(pooled + b_ref[...])


def kernel(inputs, table, W, b):
    batch, seq = inputs.shape
    vocab, dim = table.shape
    flat = seq * dim
    flatp = ((flat + _LANES - 1) // _LANES) * _LANES
    block_b = 512
    n_chunks = 8
    chunk = batch // n_chunks

    j = jnp.arange(flatp, dtype=jnp.int32)
    s = jnp.arange(seq, dtype=jnp.int32)
    rep_mat = (s[:, None] == (j[None, :] // dim)).astype(jnp.bfloat16)
    src_row = jnp.pad(table.reshape(-1), (0, _LANES - vocab * dim))[None, :]

    call = pl.pallas_call(
        functools.partial(_expand_body, seq=seq, dim=dim),
        grid=(chunk // block_b,),
        in_specs=[
            pl.BlockSpec((block_b, seq), lambda i: (i, 0)),
            pl.BlockSpec((seq, flatp), lambda i: (0, 0)),
            pl.BlockSpec((1, _LANES), lambda i: (0, 0)),
            pl.BlockSpec((dim, dim), lambda i: (0, 0)),
            pl.BlockSpec((1, dim), lambda i: (0, 0)),
        ],
        out_specs=[
            pl.BlockSpec((block_b, flat), lambda i: (i, 0)),
            pl.BlockSpec((block_b, dim), lambda i: (i, 0)),
        ],
        out_shape=[
            jax.ShapeDtypeStruct((chunk, flat), jnp.float32),
            jax.ShapeDtypeStruct((chunk, dim), jnp.float32),
        ],
    )
    seq_parts, pooled_parts = [], []
    for c in range(n_chunks):
        of, po = call(inputs[c * chunk:(c + 1) * chunk], rep_mat, src_row,
                      W, b[None, :])
        seq_parts.append(of.reshape(chunk, seq, dim))
        pooled_parts.append(po)
    return (jnp.concatenate(seq_parts, axis=0),
            jnp.concatenate(pooled_parts, axis=0))
